# Initial kernel scaffold; baseline (speedup 1.0000x reference)
import jax
import jax.numpy as jnp
from jax.experimental import pallas as pl


def kernel(x, edge_index, batch, emb, W1, b1, W2, b2, Wg, bg, Wc, bc):
    raise NotImplementedError("write your pallas kernel here")


# speedup vs baseline: 9.9858x; 9.9858x over previous
import functools

import jax
import jax.numpy as jnp
from jax import lax
from jax.experimental import pallas as pl
from jax.experimental.pallas import tpu as pltpu
from jax.experimental.pallas import tpu_sc as plsc

N = 50000
E = 800000
V = 100000
D = 128
H = 64
B = 64

NC = 2
NS = 16
CH = 128

E_PAD = 802816
ACC_ROWS = 50048
ROWS_PER_TILE_INIT = ACC_ROWS // NS
ROWS_PER_TILE_OUT = ACC_ROWS // NS
GN_PAD = 53248

f32 = jnp.float32
i32 = jnp.int32


def _sc_deg_body(didx_hbm, zeros_hbm, out_hbm, ones_v, idxd_v, acc):
    c = lax.axis_index("c")
    s = lax.axis_index("s")

    def fill(i, carry):
        ones_v[i] = jnp.ones((16,), f32)
        return carry

    lax.fori_loop(0, CH, fill, 0)
    pltpu.sync_copy(
        zeros_hbm.at[pl.ds(s * ROWS_PER_TILE_INIT, ROWS_PER_TILE_INIT)],
        acc.at[pl.ds(s * ROWS_PER_TILE_INIT, ROWS_PER_TILE_INIT)],
    )
    plsc.subcore_barrier()

    base0 = c * (E_PAD // 2) + s * (E_PAD // (2 * NS))

    def body(j, carry):
        base = base0 + j * CH
        pltpu.sync_copy(didx_hbm.at[pl.ds(base, CH)], idxd_v)
        pltpu.sync_copy(ones_v, acc.at[idxd_v], add=True)
        return carry

    lax.fori_loop(0, E_PAD // (2 * NS * CH), body, 0)
    plsc.subcore_barrier()
    pltpu.sync_copy(
        acc.at[pl.ds(s * ROWS_PER_TILE_OUT, ROWS_PER_TILE_OUT)],
        out_hbm.at[pl.ds(c * ACC_ROWS + s * ROWS_PER_TILE_OUT, ROWS_PER_TILE_OUT)],
    )


def _sc_gather_body(tab_hbm, xidx_hbm, out_hbm, idx_v, rows_v, sem):
    c = lax.axis_index("c")
    s = lax.axis_index("s")
    wid = s * NC + c
    per_tile = GN_PAD // (NC * NS)

    def body(j, carry):
        base = wid * per_tile + j * CH
        pltpu.sync_copy(xidx_hbm.at[pl.ds(base, CH)], idx_v)
        pltpu.async_copy(tab_hbm.at[idx_v], rows_v, sem).wait()
        pltpu.sync_copy(rows_v, out_hbm.at[pl.ds(base, CH)])
        return carry

    lax.fori_loop(0, per_tile // CH, body, 0)


def _sc_agg_body(u2_hbm, sidx_hbm, didx_hbm, zeros_hbm, out_hbm,
                 idxs_v, idxd_v, rows_v, acc, sem):
    c = lax.axis_index("c")
    s = lax.axis_index("s")

    pltpu.sync_copy(
        zeros_hbm.at[pl.ds(s * ROWS_PER_TILE_INIT, ROWS_PER_TILE_INIT)],
        acc.at[pl.ds(s * ROWS_PER_TILE_INIT, ROWS_PER_TILE_INIT)],
    )
    plsc.subcore_barrier()

    per_tile = E_PAD // NS
    base_s = c * E_PAD + s * per_tile
    base_d = s * per_tile

    def body(j, carry):
        off = j * CH
        pltpu.sync_copy(sidx_hbm.at[pl.ds(base_s + off, CH)], idxs_v)
        pltpu.sync_copy(didx_hbm.at[pl.ds(base_d + off, CH)], idxd_v)
        pltpu.async_copy(u2_hbm.at[idxs_v], rows_v, sem).wait()
        pltpu.sync_copy(rows_v, acc.at[idxd_v], add=True)
        return carry

    lax.fori_loop(0, per_tile // CH, body, 0)
    plsc.subcore_barrier()
    pltpu.sync_copy(
        acc.at[pl.ds(s * ROWS_PER_TILE_OUT, ROWS_PER_TILE_OUT)],
        out_hbm.at[pl.ds(c * ACC_ROWS + s * ROWS_PER_TILE_OUT, ROWS_PER_TILE_OUT)],
    )


@functools.cache
def _sc_kernels():
    mesh = plsc.VectorSubcoreMesh(core_axis_name="c", subcore_axis_name="s")
    params = pltpu.CompilerParams(use_tc_tiling_on_sc=False)
    deg = pl.kernel(
        _sc_deg_body,
        out_type=jax.ShapeDtypeStruct((NC * ACC_ROWS, 16), f32),
        mesh=mesh,
        compiler_params=params,
        scratch_types=[
            pltpu.VMEM((CH, 16), f32),
            pltpu.VMEM((CH,), i32),
            pltpu.VMEM_SHARED((ACC_ROWS, 16), f32),
        ],
    )
    gather = pl.kernel(
        _sc_gather_body,
        out_type=jax.ShapeDtypeStruct((GN_PAD, H), f32),
        mesh=mesh,
        compiler_params=params,
        scratch_types=[
            pltpu.VMEM((CH,), i32),
            pltpu.VMEM((CH, H), f32),
            pltpu.SemaphoreType.DMA,
        ],
    )
    agg = pl.kernel(
        _sc_agg_body,
        out_type=jax.ShapeDtypeStruct((NC * ACC_ROWS, H // 2), f32),
        mesh=mesh,
        compiler_params=params,
        scratch_types=[
            pltpu.VMEM((CH,), i32),
            pltpu.VMEM((CH,), i32),
            pltpu.VMEM((CH, H // 2), f32),
            pltpu.VMEM_SHARED((ACC_ROWS, H // 2), f32),
            pltpu.SemaphoreType.DMA,
        ],
    )
    return deg, gather, agg


_NB = 50
_BLK = N // _NB


def _mm_body(a_ref, w_ref, o_ref):
    o_ref[...] = jnp.dot(a_ref[...], w_ref[...], preferred_element_type=f32)


def _tc_table_mm(emb, W1):
    nb = V // _BLK
    return pl.pallas_call(
        _mm_body,
        grid=(nb,),
        in_specs=[
            pl.BlockSpec((_BLK, D), lambda i: (i, 0)),
            pl.BlockSpec((D, H), lambda i: (0, 0)),
        ],
        out_specs=pl.BlockSpec((_BLK, H), lambda i: (i, 0)),
        out_shape=jax.ShapeDtypeStruct((V, H), f32),
    )(emb, W1)


def _scale_body(d0_ref, d1_ref, h0_ref, dinv_ref, u_ref):
    dv = lax.rsqrt(d0_ref[...] + d1_ref[...] + 1.0)
    dinv_ref[...] = dv
    u_ref[...] = h0_ref[...] * dv


def _tc_scale(d0, d1, h0):
    return pl.pallas_call(
        _scale_body,
        grid=(_NB,),
        in_specs=[
            pl.BlockSpec((_BLK, 1), lambda i: (i, 0)),
            pl.BlockSpec((_BLK, 1), lambda i: (i, 0)),
            pl.BlockSpec((_BLK, H), lambda i: (i, 0)),
        ],
        out_specs=[
            pl.BlockSpec((_BLK, 1), lambda i: (i, 0)),
            pl.BlockSpec((_BLK, H), lambda i: (i, 0)),
        ],
        out_shape=[
            jax.ShapeDtypeStruct((N, 1), f32),
            jax.ShapeDtypeStruct((N, H), f32),
        ],
    )(d0, d1, h0)


def _layer2_body(agg_ref, u_ref, dinv_ref, b1_ref, w2_ref, u2_ref):
    h1 = jax.nn.relu(dinv_ref[...] * (agg_ref[...] + u_ref[...]) + b1_ref[...])
    u2_ref[...] = jnp.dot(h1, w2_ref[...], preferred_element_type=f32) * dinv_ref[...]


def _tc_layer2(agg1, U1, dinv, b1, W2):
    return pl.pallas_call(
        _layer2_body,
        grid=(_NB,),
        in_specs=[
            pl.BlockSpec((_BLK, H), lambda i: (i, 0)),
            pl.BlockSpec((_BLK, H), lambda i: (i, 0)),
            pl.BlockSpec((_BLK, 1), lambda i: (i, 0)),
            pl.BlockSpec((1, H), lambda i: (0, 0)),
            pl.BlockSpec((H, H), lambda i: (0, 0)),
        ],
        out_specs=pl.BlockSpec((_BLK, H), lambda i: (i, 0)),
        out_shape=jax.ShapeDtypeStruct((N, H), f32),
    )(agg1, U1, dinv, b1, W2)


def _gate_body(agg_ref, u_ref, dinv_ref, b2_ref, wg_ref, bg_ref, batch_ref,
               h2_ref, g_ref, gmax_ref):
    i = pl.program_id(0)
    h2 = jax.nn.relu(dinv_ref[...] * (agg_ref[...] + u_ref[...]) + b2_ref[...])
    g = jnp.dot(h2, wg_ref[...], preferred_element_type=f32) + bg_ref[...]
    oh = lax.broadcasted_iota(i32, (_BLK, B), 1) == batch_ref[...]
    gm = jnp.where(oh, g, -jnp.inf)
    bmax = jnp.max(gm, axis=0, keepdims=True)

    @pl.when(i == 0)
    def _():
        gmax_ref[...] = jnp.full((1, B), -jnp.inf, f32)

    gmax_ref[...] = jnp.maximum(gmax_ref[...], bmax)
    h2_ref[...] = h2
    g_ref[...] = g


def _tc_gate(agg2, U2, dinv, b2, Wg, bg, batch2d):
    return pl.pallas_call(
        _gate_body,
        grid=(_NB,),
        in_specs=[
            pl.BlockSpec((_BLK, H), lambda i: (i, 0)),
            pl.BlockSpec((_BLK, H), lambda i: (i, 0)),
            pl.BlockSpec((_BLK, 1), lambda i: (i, 0)),
            pl.BlockSpec((1, H), lambda i: (0, 0)),
            pl.BlockSpec((H, 1), lambda i: (0, 0)),
            pl.BlockSpec((1, 1), lambda i: (0, 0)),
            pl.BlockSpec((_BLK, 1), lambda i: (i, 0)),
        ],
        out_specs=[
            pl.BlockSpec((_BLK, H), lambda i: (i, 0)),
            pl.BlockSpec((_BLK, 1), lambda i: (i, 0)),
            pl.BlockSpec((1, B), lambda i: (0, 0)),
        ],
        out_shape=[
            jax.ShapeDtypeStruct((N, H), f32),
            jax.ShapeDtypeStruct((N, 1), f32),
            jax.ShapeDtypeStruct((1, B), f32),
        ],
    )(agg2, U2, dinv, b2, Wg, bg, batch2d)


def _pool_body(h2_ref, g_ref, batch_ref, gmax_ref, wc_ref, bc_ref, res_ref,
               num_s, den_s):
    i = pl.program_id(0)

    @pl.when(i == 0)
    def _():
        num_s[...] = jnp.zeros((B, H), f32)
        den_s[...] = jnp.zeros((B, 1), f32)

    oh = (lax.broadcasted_iota(i32, (_BLK, B), 1) == batch_ref[...]).astype(f32)
    gmax = gmax_ref[...]
    gm0 = jnp.where(jnp.isfinite(gmax), gmax, 0.0)
    gnode = lax.dot_general(oh, gm0, (((1,), (1,)), ((), ())),
                            preferred_element_type=f32)
    e = jnp.exp(g_ref[...] - gnode)
    den_s[...] += lax.dot_general(oh, e, (((0,), (0,)), ((), ())),
                                  preferred_element_type=f32)
    num_s[...] += lax.dot_general(oh, e * h2_ref[...], (((0,), (0,)), ((), ())),
                                  preferred_element_type=f32)

    @pl.when(i == _NB - 1)
    def _():
        nv = jnp.dot(num_s[...], wc_ref[...], preferred_element_type=f32)
        res_ref[...] = nv / jnp.maximum(den_s[...], 1e-16) + bc_ref[...]


def _tc_pool(h2, g, batch2d, gmax, Wc, bc):
    return pl.pallas_call(
        _pool_body,
        grid=(_NB,),
        in_specs=[
            pl.BlockSpec((_BLK, H), lambda i: (i, 0)),
            pl.BlockSpec((_BLK, 1), lambda i: (i, 0)),
            pl.BlockSpec((_BLK, 1), lambda i: (i, 0)),
            pl.BlockSpec((1, B), lambda i: (0, 0)),
            pl.BlockSpec((H, 1), lambda i: (0, 0)),
            pl.BlockSpec((1, 1), lambda i: (0, 0)),
        ],
        out_specs=pl.BlockSpec((B, 1), lambda i: (0, 0)),
        out_shape=jax.ShapeDtypeStruct((B, 1), f32),
        scratch_shapes=[
            pltpu.VMEM((B, H), f32),
            pltpu.VMEM((B, 1), f32),
        ],
    )(h2, g, batch2d, gmax, Wc, bc)


def _interleave_pad(U):
    u2 = U.reshape(2 * N, H // 2)
    return jnp.concatenate([u2, jnp.zeros((16, H // 2), f32)], axis=0)


def kernel(x, edge_index, batch, emb, W1, b1, W2, b2, Wg, bg, Wc, bc):
    _sc_deg, _sc_gather, _sc_agg = _sc_kernels()
    src = edge_index[0].astype(i32)
    dst = edge_index[1].astype(i32)

    pad_e = E_PAD - E
    src_p = jnp.concatenate([src, jnp.full((pad_e,), N, i32)])
    dst_p = jnp.concatenate([dst, jnp.full((pad_e,), N, i32)])
    sidx = jnp.concatenate([2 * src_p, 2 * src_p + 1])
    x_p = jnp.concatenate([x.astype(i32), jnp.zeros((GN_PAD - N,), i32)])
    zeros16 = jnp.zeros((ACC_ROWS, 16), f32)
    zeros32 = jnp.zeros((ACC_ROWS, H // 2), f32)

    T1 = _tc_table_mm(emb, W1)
    h0 = _sc_gather(T1, x_p)[:N]

    dpart = _sc_deg(dst_p, zeros16)
    dinv, U1 = _tc_scale(dpart[:N, :1], dpart[ACC_ROWS:ACC_ROWS + N, :1], h0)

    a1 = _sc_agg(_interleave_pad(U1), sidx, dst_p, zeros32)
    agg1 = jnp.concatenate([a1[:N], a1[ACC_ROWS:ACC_ROWS + N]], axis=1)
    U2 = _tc_layer2(agg1, U1, dinv, b1.reshape(1, H), W2)
    a2 = _sc_agg(_interleave_pad(U2), sidx, dst_p, zeros32)
    agg2 = jnp.concatenate([a2[:N], a2[ACC_ROWS:ACC_ROWS + N]], axis=1)

    batch2d = batch.astype(i32).reshape(N, 1)
    h2, g, gmax = _tc_gate(agg2, U2, dinv, b2.reshape(1, H), Wg,
                           bg.reshape(1, 1), batch2d)
    return _tc_pool(h2, g, batch2d, gmax, Wc, bc.reshape(1, 1))


# speedup vs baseline: 15.3389x; 1.5361x over previous
import functools

import jax
import jax.numpy as jnp
from jax import lax
from jax.experimental import pallas as pl
from jax.experimental.pallas import tpu as pltpu
from jax.experimental.pallas import tpu_sc as plsc

N = 50000
E = 800000
V = 100000
D = 128
H = 64
B = 64

NC = 2
NS = 16
CH = 128

E_PAD = 802816
ACC_ROWS = 50048
ROWS_PER_TILE_INIT = ACC_ROWS // NS
ROWS_PER_TILE_OUT = ACC_ROWS // NS
GN_PAD = 53248

f32 = jnp.float32
i32 = jnp.int32


def _sc_deg_body(didx_hbm, zeros_hbm, out_hbm, ones_v, idxd_v, acc):
    c = lax.axis_index("c")
    s = lax.axis_index("s")

    def fill(i, carry):
        ones_v[i] = jnp.ones((16,), f32)
        return carry

    lax.fori_loop(0, CH, fill, 0)
    pltpu.sync_copy(
        zeros_hbm.at[pl.ds(s * ROWS_PER_TILE_INIT, ROWS_PER_TILE_INIT)],
        acc.at[pl.ds(s * ROWS_PER_TILE_INIT, ROWS_PER_TILE_INIT)],
    )
    plsc.subcore_barrier()

    base0 = c * (E_PAD // 2) + s * (E_PAD // (2 * NS))

    def body(j, carry):
        base = base0 + j * CH
        pltpu.sync_copy(didx_hbm.at[pl.ds(base, CH)], idxd_v)
        pltpu.sync_copy(ones_v, acc.at[idxd_v], add=True)
        return carry

    lax.fori_loop(0, E_PAD // (2 * NS * CH), body, 0)
    plsc.subcore_barrier()
    pltpu.sync_copy(
        acc.at[pl.ds(s * ROWS_PER_TILE_OUT, ROWS_PER_TILE_OUT)],
        out_hbm.at[pl.ds(c * ACC_ROWS + s * ROWS_PER_TILE_OUT, ROWS_PER_TILE_OUT)],
    )


def _sc_gather_body(tab_hbm, xidx_hbm, out_hbm, idx_v, rows_v, sem):
    c = lax.axis_index("c")
    s = lax.axis_index("s")
    wid = s * NC + c
    per_tile = GN_PAD // (NC * NS)

    def body(j, carry):
        base = wid * per_tile + j * CH
        pltpu.sync_copy(xidx_hbm.at[pl.ds(base, CH)], idx_v)
        pltpu.async_copy(tab_hbm.at[idx_v], rows_v, sem).wait()
        pltpu.sync_copy(rows_v, out_hbm.at[pl.ds(base, CH)])
        return carry

    lax.fori_loop(0, per_tile // CH, body, 0)


KB = 4
CHUNKS_PER_TILE = E_PAD // (NS * CH)
BATCHES_PER_TILE = CHUNKS_PER_TILE // KB


def _sc_agg_body(u2_hbm, sidx_hbm, didx_hbm, zeros_hbm, out_hbm,
                 sidx_v, didx_v, rows_v, acc, semg, sems):
    c = lax.axis_index("c")
    s = lax.axis_index("s")

    pltpu.sync_copy(
        zeros_hbm.at[pl.ds(s * ROWS_PER_TILE_INIT, ROWS_PER_TILE_INIT)],
        acc.at[pl.ds(s * ROWS_PER_TILE_INIT, ROWS_PER_TILE_INIT)],
    )
    plsc.subcore_barrier()

    srow0 = (c * E_PAD) // CH + s * CHUNKS_PER_TILE
    drow0 = s * CHUNKS_PER_TILE

    def body(i, carry):
        pltpu.sync_copy(sidx_hbm.at[pl.ds(srow0 + i * KB, KB)], sidx_v)
        pltpu.sync_copy(didx_hbm.at[pl.ds(drow0 + i * KB, KB)], didx_v)
        gathers = [
            pltpu.async_copy(u2_hbm.at[sidx_v.at[j]],
                             rows_v.at[pl.ds(j * CH, CH)], semg)
            for j in range(KB)
        ]
        for j in range(KB):
            gathers[j].wait()
            pltpu.sync_copy(rows_v.at[pl.ds(j * CH, CH)],
                            acc.at[didx_v.at[j]], add=True)
        return carry

    lax.fori_loop(0, BATCHES_PER_TILE, body, 0)
    plsc.subcore_barrier()
    pltpu.sync_copy(
        acc.at[pl.ds(s * ROWS_PER_TILE_OUT, ROWS_PER_TILE_OUT)],
        out_hbm.at[pl.ds(c * ACC_ROWS + s * ROWS_PER_TILE_OUT, ROWS_PER_TILE_OUT)],
    )


@functools.cache
def _sc_kernels():
    mesh = plsc.VectorSubcoreMesh(core_axis_name="c", subcore_axis_name="s")
    params = pltpu.CompilerParams(use_tc_tiling_on_sc=False)
    deg = pl.kernel(
        _sc_deg_body,
        out_type=jax.ShapeDtypeStruct((NC * ACC_ROWS, 16), f32),
        mesh=mesh,
        compiler_params=params,
        scratch_types=[
            pltpu.VMEM((CH, 16), f32),
            pltpu.VMEM((CH,), i32),
            pltpu.VMEM_SHARED((ACC_ROWS, 16), f32),
        ],
    )
    gather = pl.kernel(
        _sc_gather_body,
        out_type=jax.ShapeDtypeStruct((GN_PAD, H), f32),
        mesh=mesh,
        compiler_params=params,
        scratch_types=[
            pltpu.VMEM((CH,), i32),
            pltpu.VMEM((CH, H), f32),
            pltpu.SemaphoreType.DMA,
        ],
    )
    agg = pl.kernel(
        _sc_agg_body,
        out_type=jax.ShapeDtypeStruct((NC * ACC_ROWS, H // 2), f32),
        mesh=mesh,
        compiler_params=params,
        scratch_types=[
            pltpu.VMEM((KB, CH), i32),
            pltpu.VMEM((KB, CH), i32),
            pltpu.VMEM((KB * CH, H // 2), f32),
            pltpu.VMEM_SHARED((ACC_ROWS, H // 2), f32),
            pltpu.SemaphoreType.DMA,
            pltpu.SemaphoreType.DMA,
        ],
    )
    return deg, gather, agg


_NB = 50
_BLK = N // _NB


def _mm_body(a_ref, w_ref, o_ref):
    o_ref[...] = jnp.dot(a_ref[...], w_ref[...], preferred_element_type=f32)


def _tc_table_mm(emb, W1):
    nb = V // _BLK
    return pl.pallas_call(
        _mm_body,
        grid=(nb,),
        in_specs=[
            pl.BlockSpec((_BLK, D), lambda i: (i, 0)),
            pl.BlockSpec((D, H), lambda i: (0, 0)),
        ],
        out_specs=pl.BlockSpec((_BLK, H), lambda i: (i, 0)),
        out_shape=jax.ShapeDtypeStruct((V, H), f32),
    )(emb, W1)


def _scale_body(d0_ref, d1_ref, h0_ref, dinv_ref, u_ref):
    dv = lax.rsqrt(d0_ref[...] + d1_ref[...] + 1.0)
    dinv_ref[...] = dv
    u_ref[...] = h0_ref[...] * dv


def _tc_scale(d0, d1, h0):
    return pl.pallas_call(
        _scale_body,
        grid=(_NB,),
        in_specs=[
            pl.BlockSpec((_BLK, 1), lambda i: (i, 0)),
            pl.BlockSpec((_BLK, 1), lambda i: (i, 0)),
            pl.BlockSpec((_BLK, H), lambda i: (i, 0)),
        ],
        out_specs=[
            pl.BlockSpec((_BLK, 1), lambda i: (i, 0)),
            pl.BlockSpec((_BLK, H), lambda i: (i, 0)),
        ],
        out_shape=[
            jax.ShapeDtypeStruct((N, 1), f32),
            jax.ShapeDtypeStruct((N, H), f32),
        ],
    )(d0, d1, h0)


def _layer2_body(agg_ref, u_ref, dinv_ref, b1_ref, w2_ref, u2_ref):
    h1 = jax.nn.relu(dinv_ref[...] * (agg_ref[...] + u_ref[...]) + b1_ref[...])
    u2_ref[...] = jnp.dot(h1, w2_ref[...], preferred_element_type=f32) * dinv_ref[...]


def _tc_layer2(agg1, U1, dinv, b1, W2):
    return pl.pallas_call(
        _layer2_body,
        grid=(_NB,),
        in_specs=[
            pl.BlockSpec((_BLK, H), lambda i: (i, 0)),
            pl.BlockSpec((_BLK, H), lambda i: (i, 0)),
            pl.BlockSpec((_BLK, 1), lambda i: (i, 0)),
            pl.BlockSpec((1, H), lambda i: (0, 0)),
            pl.BlockSpec((H, H), lambda i: (0, 0)),
        ],
        out_specs=pl.BlockSpec((_BLK, H), lambda i: (i, 0)),
        out_shape=jax.ShapeDtypeStruct((N, H), f32),
    )(agg1, U1, dinv, b1, W2)


def _gate_body(agg_ref, u_ref, dinv_ref, b2_ref, wg_ref, bg_ref, batch_ref,
               h2_ref, g_ref, gmax_ref):
    i = pl.program_id(0)
    h2 = jax.nn.relu(dinv_ref[...] * (agg_ref[...] + u_ref[...]) + b2_ref[...])
    g = jnp.dot(h2, wg_ref[...], preferred_element_type=f32) + bg_ref[...]
    oh = lax.broadcasted_iota(i32, (_BLK, B), 1) == batch_ref[...]
    gm = jnp.where(oh, g, -jnp.inf)
    bmax = jnp.max(gm, axis=0, keepdims=True)

    @pl.when(i == 0)
    def _():
        gmax_ref[...] = jnp.full((1, B), -jnp.inf, f32)

    gmax_ref[...] = jnp.maximum(gmax_ref[...], bmax)
    h2_ref[...] = h2
    g_ref[...] = g


def _tc_gate(agg2, U2, dinv, b2, Wg, bg, batch2d):
    return pl.pallas_call(
        _gate_body,
        grid=(_NB,),
        in_specs=[
            pl.BlockSpec((_BLK, H), lambda i: (i, 0)),
            pl.BlockSpec((_BLK, H), lambda i: (i, 0)),
            pl.BlockSpec((_BLK, 1), lambda i: (i, 0)),
            pl.BlockSpec((1, H), lambda i: (0, 0)),
            pl.BlockSpec((H, 1), lambda i: (0, 0)),
            pl.BlockSpec((1, 1), lambda i: (0, 0)),
            pl.BlockSpec((_BLK, 1), lambda i: (i, 0)),
        ],
        out_specs=[
            pl.BlockSpec((_BLK, H), lambda i: (i, 0)),
            pl.BlockSpec((_BLK, 1), lambda i: (i, 0)),
            pl.BlockSpec((1, B), lambda i: (0, 0)),
        ],
        out_shape=[
            jax.ShapeDtypeStruct((N, H), f32),
            jax.ShapeDtypeStruct((N, 1), f32),
            jax.ShapeDtypeStruct((1, B), f32),
        ],
    )(agg2, U2, dinv, b2, Wg, bg, batch2d)


def _pool_body(h2_ref, g_ref, batch_ref, gmax_ref, wc_ref, bc_ref, res_ref,
               num_s, den_s):
    i = pl.program_id(0)

    @pl.when(i == 0)
    def _():
        num_s[...] = jnp.zeros((B, H), f32)
        den_s[...] = jnp.zeros((B, 1), f32)

    oh = (lax.broadcasted_iota(i32, (_BLK, B), 1) == batch_ref[...]).astype(f32)
    gmax = gmax_ref[...]
    gm0 = jnp.where(jnp.isfinite(gmax), gmax, 0.0)
    gnode = lax.dot_general(oh, gm0, (((1,), (1,)), ((), ())),
                            preferred_element_type=f32)
    e = jnp.exp(g_ref[...] - gnode)
    den_s[...] += lax.dot_general(oh, e, (((0,), (0,)), ((), ())),
                                  preferred_element_type=f32)
    num_s[...] += lax.dot_general(oh, e * h2_ref[...], (((0,), (0,)), ((), ())),
                                  preferred_element_type=f32)

    @pl.when(i == _NB - 1)
    def _():
        nv = jnp.dot(num_s[...], wc_ref[...], preferred_element_type=f32)
        res_ref[...] = nv / jnp.maximum(den_s[...], 1e-16) + bc_ref[...]


def _tc_pool(h2, g, batch2d, gmax, Wc, bc):
    return pl.pallas_call(
        _pool_body,
        grid=(_NB,),
        in_specs=[
            pl.BlockSpec((_BLK, H), lambda i: (i, 0)),
            pl.BlockSpec((_BLK, 1), lambda i: (i, 0)),
            pl.BlockSpec((_BLK, 1), lambda i: (i, 0)),
            pl.BlockSpec((1, B), lambda i: (0, 0)),
            pl.BlockSpec((H, 1), lambda i: (0, 0)),
            pl.BlockSpec((1, 1), lambda i: (0, 0)),
        ],
        out_specs=pl.BlockSpec((B, 1), lambda i: (0, 0)),
        out_shape=jax.ShapeDtypeStruct((B, 1), f32),
        scratch_shapes=[
            pltpu.VMEM((B, H), f32),
            pltpu.VMEM((B, 1), f32),
        ],
    )(h2, g, batch2d, gmax, Wc, bc)


def _interleave_pad(U):
    u2 = U.reshape(2 * N, H // 2)
    return jnp.concatenate([u2, jnp.zeros((16, H // 2), f32)], axis=0)


def kernel(x, edge_index, batch, emb, W1, b1, W2, b2, Wg, bg, Wc, bc):
    _sc_deg, _sc_gather, _sc_agg = _sc_kernels()
    src = edge_index[0].astype(i32)
    dst = edge_index[1].astype(i32)

    pad_e = E_PAD - E
    src_p = jnp.concatenate([src, jnp.full((pad_e,), N, i32)])
    dst_p = jnp.concatenate([dst, jnp.full((pad_e,), N, i32)])
    sidx = jnp.concatenate([2 * src_p, 2 * src_p + 1])
    x_p = jnp.concatenate([x.astype(i32), jnp.zeros((GN_PAD - N,), i32)])
    zeros16 = jnp.zeros((ACC_ROWS, 16), f32)
    zeros32 = jnp.zeros((ACC_ROWS, H // 2), f32)

    T1 = _tc_table_mm(emb, W1)
    h0 = _sc_gather(T1, x_p)[:N]

    dpart = _sc_deg(dst_p, zeros16)
    dinv, U1 = _tc_scale(dpart[:N, :1], dpart[ACC_ROWS:ACC_ROWS + N, :1], h0)

    sidx2 = sidx.reshape(2 * E_PAD // CH, CH)
    didx2 = dst_p.reshape(E_PAD // CH, CH)
    a1 = _sc_agg(_interleave_pad(U1), sidx2, didx2, zeros32)
    agg1 = jnp.concatenate([a1[:N], a1[ACC_ROWS:ACC_ROWS + N]], axis=1)
    U2 = _tc_layer2(agg1, U1, dinv, b1.reshape(1, H), W2)
    a2 = _sc_agg(_interleave_pad(U2), sidx2, didx2, zeros32)
    agg2 = jnp.concatenate([a2[:N], a2[ACC_ROWS:ACC_ROWS + N]], axis=1)

    batch2d = batch.astype(i32).reshape(N, 1)
    h2, g, gmax = _tc_gate(agg2, U2, dinv, b2.reshape(1, H), Wg,
                           bg.reshape(1, 1), batch2d)
    return _tc_pool(h2, g, batch2d, gmax, Wc, bc.reshape(1, 1))
